# baseline (speedup 1.0000x reference)
import jax
import jax.numpy as jnp
from jax import lax
from jax.experimental import pallas as pl
from jax.experimental.pallas import tpu as pltpu
from jax.experimental.pallas import tpu_sc as plsc

N = 10000
E = 320000
D = 128
NC = 2
NS = 16
NW = NC * NS
BLK = 128
NBLK = 81
EPT = NBLK * BLK
EPAD = NW * EPT
NP = 10016
RPT = NP // NS
DEGW = 16
BR = 2000


def _make_agg(with_deg):
    mesh = plsc.VectorSubcoreMesh(core_axis_name="c", subcore_axis_name="s")
    out_type = [jax.ShapeDtypeStruct((NC, NP, D), jnp.float32)]
    scratch = (
        [pltpu.VMEM((BLK,), jnp.int32) for _ in range(4)]
        + [pltpu.VMEM((BLK,), jnp.int32) for _ in range(4)]
        + [
            pltpu.VMEM((BLK, D), jnp.float32),
            pltpu.VMEM((BLK, D), jnp.float32),
            pltpu.VMEM_SHARED((NP, D), jnp.float32),
        ]
        + [pltpu.SemaphoreType.DMA for _ in range(4)]
        + [pltpu.SemaphoreType.DMA for _ in range(2)]
    )
    if with_deg:
        out_type.append(jax.ShapeDtypeStruct((NC, NP, DEGW), jnp.float32))
        scratch += [
            pltpu.VMEM((BLK, DEGW), jnp.float32),
            pltpu.VMEM((BLK, DEGW), jnp.float32),
            pltpu.VMEM_SHARED((NP, DEGW), jnp.float32),
        ]

    assert NBLK % 4 == 1 and NBLK >= 5
    NFULL, NREM = RPT // BLK, RPT % BLK

    def body(*refs):
        if with_deg:
            (table, edges, out_p, out_d, *rest) = refs
            (s0, s1_, s2_, s3_, d0, d1_, d2_, d3_, rows0, rows1, acc_sh,
             is0, is1, is2, is3, gs0, gs1, ones_v, zd_v, deg_sh) = rest
        else:
            (table, edges, out_p, *rest) = refs
            (s0, s1_, s2_, s3_, d0, d1_, d2_, d3_, rows0, rows1, acc_sh,
             is0, is1, is2, is3, gs0, gs1) = rest
        sib = (s0, s1_, s2_, s3_)
        dib = (d0, d1_, d2_, d3_)
        isem = (is0, is1, is2, is3)
        c = lax.axis_index("c")
        s = lax.axis_index("s")
        wid = s * NC + c
        e0 = wid * EPT
        r0 = s * RPT

        zval = jnp.zeros((16,), jnp.float32)

        def zrow(i, carry):
            for kk in range(D // 16):
                rows0[i, pl.ds(16 * kk, 16)] = zval
            if with_deg:
                ones_v[i, :] = jnp.ones((16,), jnp.float32)
                zd_v[i, :] = zval
            return carry

        lax.fori_loop(0, BLK, zrow, 0)
        for t in range(NFULL):
            pltpu.sync_copy(rows0, acc_sh.at[pl.ds(r0 + t * BLK, BLK)])
            if with_deg:
                pltpu.sync_copy(zd_v, deg_sh.at[pl.ds(r0 + t * BLK, BLK)])
        if NREM:
            pltpu.sync_copy(rows0.at[pl.ds(0, NREM)],
                            acc_sh.at[pl.ds(r0 + NFULL * BLK, NREM)])
            if with_deg:
                pltpu.sync_copy(zd_v.at[pl.ds(0, NREM)],
                                deg_sh.at[pl.ds(r0 + NFULL * BLK, NREM)])

        def icopy(j, k):
            pltpu.async_copy(edges.at[0, pl.ds(e0 + j * BLK, BLK)],
                             sib[k], isem[k])
            pltpu.async_copy(edges.at[1, pl.ds(e0 + j * BLK, BLK)],
                             dib[k], isem[k])

        def iwait(j, k):
            pltpu.make_async_copy(edges.at[0, pl.ds(e0 + j * BLK, BLK)],
                                  sib[k], isem[k]).wait()
            pltpu.make_async_copy(edges.at[1, pl.ds(e0 + j * BLK, BLK)],
                                  dib[k], isem[k]).wait()

        def gather(k, buf, sem):
            pltpu.async_copy(table.at[sib[k]], buf, sem)

        def gwait(k, buf, sem):
            pltpu.make_async_copy(table.at[sib[k]], buf, sem).wait()

        def scat(k, buf):
            pltpu.sync_copy(buf, acc_sh.at[dib[k]], add=True)
            if with_deg:
                pltpu.sync_copy(ones_v, deg_sh.at[dib[k]], add=True)

        pltpu.sync_copy(edges.at[0, pl.ds(e0, BLK)], sib[0])
        pltpu.sync_copy(edges.at[1, pl.ds(e0, BLK)], dib[0])
        gather(0, rows0, gs0)
        icopy(1, 1)
        icopy(2, 2)
        icopy(3, 3)
        plsc.subcore_barrier()

        def quad(jj, carry):
            b0 = 4 * jj
            iwait(b0 + 1, 1)
            gather(1, rows1, gs1)
            gwait(0, rows0, gs0)
            scat(0, rows0)
            icopy(b0 + 4, 0)

            iwait(b0 + 2, 2)
            gather(2, rows0, gs0)
            gwait(1, rows1, gs1)
            scat(1, rows1)

            @pl.when(b0 + 5 < NBLK)
            def _():
                icopy(b0 + 5, 1)

            iwait(b0 + 3, 3)
            gather(3, rows1, gs1)
            gwait(2, rows0, gs0)
            scat(2, rows0)

            @pl.when(b0 + 6 < NBLK)
            def _():
                icopy(b0 + 6, 2)

            iwait(b0 + 4, 0)
            gather(0, rows0, gs0)
            gwait(3, rows1, gs1)
            scat(3, rows1)

            @pl.when(b0 + 7 < NBLK)
            def _():
                icopy(b0 + 7, 3)

            return carry

        lax.fori_loop(0, NBLK // 4, quad, 0)
        gwait(0, rows0, gs0)
        scat(0, rows0)
        plsc.subcore_barrier()
        pltpu.sync_copy(acc_sh.at[pl.ds(r0, RPT)], out_p.at[c, pl.ds(r0, RPT)])
        if with_deg:
            pltpu.sync_copy(deg_sh.at[pl.ds(r0, RPT)],
                            out_d.at[c, pl.ds(r0, RPT)])

    return pl.kernel(body, out_type=out_type, mesh=mesh,
                     scratch_types=scratch,
                     compiler_params=pltpu.CompilerParams(
                         use_tc_tiling_on_sc=False))


_agg_deg = _make_agg(True)
_agg = _make_agg(False)


def _k1_body(x_ref, wn_ref, ws_ref, b_ref, a_ref, z_ref):
    xb = x_ref[...]
    a_ref[...] = jnp.dot(xb, wn_ref[...], preferred_element_type=jnp.float32)
    z_ref[...] = (jnp.dot(xb, ws_ref[...], preferred_element_type=jnp.float32)
                  + b_ref[...])


_k1 = pl.pallas_call(
    _k1_body,
    grid=(N // BR,),
    in_specs=[
        pl.BlockSpec((BR, D), lambda i: (i, 0)),
        pl.BlockSpec((D, D), lambda i: (0, 0)),
        pl.BlockSpec((D, D), lambda i: (0, 0)),
        pl.BlockSpec((1, D), lambda i: (0, 0)),
    ],
    out_specs=[
        pl.BlockSpec((BR, D), lambda i: (i, 0)),
        pl.BlockSpec((BR, D), lambda i: (i, 0)),
    ],
    out_shape=[
        jax.ShapeDtypeStruct((N, D), jnp.float32),
        jax.ShapeDtypeStruct((N, D), jnp.float32),
    ],
)


def _k2_body(p_ref, dg_ref, z1_ref, wn_ref, ws_ref, b_ref, a2_ref, z2_ref):
    s1 = p_ref[0] + p_ref[1]
    deg = jnp.maximum(dg_ref[0, :, :1] + dg_ref[1, :, :1], 1.0)
    h = jnp.maximum(z1_ref[...] + s1 / deg, 0.0)
    a2_ref[...] = jnp.dot(h, wn_ref[...], preferred_element_type=jnp.float32)
    z2_ref[...] = (jnp.dot(h, ws_ref[...], preferred_element_type=jnp.float32)
                   + b_ref[...])


_k2 = pl.pallas_call(
    _k2_body,
    grid=(N // BR,),
    in_specs=[
        pl.BlockSpec((NC, BR, D), lambda i: (0, i, 0)),
        pl.BlockSpec((NC, BR, DEGW), lambda i: (0, i, 0)),
        pl.BlockSpec((BR, D), lambda i: (i, 0)),
        pl.BlockSpec((D, D), lambda i: (0, 0)),
        pl.BlockSpec((D, D), lambda i: (0, 0)),
        pl.BlockSpec((1, D), lambda i: (0, 0)),
    ],
    out_specs=[
        pl.BlockSpec((BR, D), lambda i: (i, 0)),
        pl.BlockSpec((BR, D), lambda i: (i, 0)),
    ],
    out_shape=[
        jax.ShapeDtypeStruct((N, D), jnp.float32),
        jax.ShapeDtypeStruct((N, D), jnp.float32),
    ],
)


def _k3_body(p2_ref, dg_ref, z2_ref, out_ref):
    s2 = p2_ref[0] + p2_ref[1]
    deg = jnp.maximum(dg_ref[0, :, :1] + dg_ref[1, :, :1], 1.0)
    out_ref[...] = z2_ref[...] + s2 / deg


_k3 = pl.pallas_call(
    _k3_body,
    grid=(N // BR,),
    in_specs=[
        pl.BlockSpec((NC, BR, D), lambda i: (0, i, 0)),
        pl.BlockSpec((NC, BR, DEGW), lambda i: (0, i, 0)),
        pl.BlockSpec((BR, D), lambda i: (i, 0)),
    ],
    out_specs=pl.BlockSpec((BR, D), lambda i: (i, 0)),
    out_shape=jax.ShapeDtypeStruct((N, D), jnp.float32),
)


def kernel(x, edge_index, W_self1, W_neigh1, b1, W_self2, W_neigh2, b2):
    ei = edge_index if edge_index.dtype == jnp.int32 else (
        edge_index.astype(jnp.int32))
    pad = jnp.concatenate(
        (jnp.zeros((1, EPAD - E), jnp.int32),
         jnp.full((1, EPAD - E), N, jnp.int32)), axis=0)
    ei = jnp.concatenate((ei, pad), axis=1)
    b1r = b1.reshape(1, D)
    b2r = b2.reshape(1, D)

    a1, z1 = _k1(x, W_neigh1, W_self1, b1r)
    p1, dg = _agg_deg(a1, ei)
    a2, z2 = _k2(p1, dg, z1, W_neigh2, W_self2, b2r)
    (p2,) = _agg(a2, ei)
    return _k3(p2, dg, z2)


# speedup vs baseline: 4.4985x; 4.4985x over previous
import jax
import jax.numpy as jnp
from jax import lax
from jax.experimental import pallas as pl
from jax.experimental.pallas import tpu as pltpu
from jax.experimental.pallas import tpu_sc as plsc

N = 10000
E = 320000
D = 128
NC = 2
NS = 16
NW = NC * NS
BLK = 80
NBLK = 125
EPT = NBLK * BLK
NP = N
RPT = NP // NS
DEGW = 16
BR = 2000


def _make_agg(with_deg):
    mesh = plsc.VectorSubcoreMesh(core_axis_name="c", subcore_axis_name="s")
    out_type = [jax.ShapeDtypeStruct((NC, NP, D), jnp.float32)]
    scratch = (
        [pltpu.VMEM((BLK,), jnp.int32) for _ in range(4)]
        + [pltpu.VMEM((BLK,), jnp.int32) for _ in range(4)]
        + [
            pltpu.VMEM((BLK, D), jnp.float32),
            pltpu.VMEM((BLK, D), jnp.float32),
            pltpu.VMEM_SHARED((NP, D), jnp.float32),
        ]
        + [pltpu.SemaphoreType.DMA for _ in range(4)]
        + [pltpu.SemaphoreType.DMA for _ in range(2)]
    )
    if with_deg:
        out_type.append(jax.ShapeDtypeStruct((NC, NP, DEGW), jnp.float32))
        scratch += [
            pltpu.VMEM((BLK, DEGW), jnp.float32),
            pltpu.VMEM((BLK, DEGW), jnp.float32),
            pltpu.VMEM_SHARED((NP, DEGW), jnp.float32),
        ]

    assert NBLK % 4 == 1 and NBLK >= 5
    NFULL, NREM = RPT // BLK, RPT % BLK

    def body(*refs):
        if with_deg:
            (table, edges, out_p, out_d, *rest) = refs
            (s0, s1_, s2_, s3_, d0, d1_, d2_, d3_, rows0, rows1, acc_sh,
             is0, is1, is2, is3, gs0, gs1, ones_v, zd_v, deg_sh) = rest
        else:
            (table, edges, out_p, *rest) = refs
            (s0, s1_, s2_, s3_, d0, d1_, d2_, d3_, rows0, rows1, acc_sh,
             is0, is1, is2, is3, gs0, gs1) = rest
        sib = (s0, s1_, s2_, s3_)
        dib = (d0, d1_, d2_, d3_)
        isem = (is0, is1, is2, is3)
        c = lax.axis_index("c")
        s = lax.axis_index("s")
        wid = s * NC + c
        e0 = wid * EPT
        r0 = s * RPT

        zval = jnp.zeros((16,), jnp.float32)

        def zrow(i, carry):
            for kk in range(D // 16):
                rows0[i, pl.ds(16 * kk, 16)] = zval
            if with_deg:
                ones_v[i, :] = jnp.ones((16,), jnp.float32)
                zd_v[i, :] = zval
            return carry

        lax.fori_loop(0, BLK, zrow, 0)
        for t in range(NFULL):
            pltpu.sync_copy(rows0, acc_sh.at[pl.ds(r0 + t * BLK, BLK)])
            if with_deg:
                pltpu.sync_copy(zd_v, deg_sh.at[pl.ds(r0 + t * BLK, BLK)])
        if NREM:
            pltpu.sync_copy(rows0.at[pl.ds(0, NREM)],
                            acc_sh.at[pl.ds(r0 + NFULL * BLK, NREM)])
            if with_deg:
                pltpu.sync_copy(zd_v.at[pl.ds(0, NREM)],
                                deg_sh.at[pl.ds(r0 + NFULL * BLK, NREM)])

        def icopy(j, k):
            pltpu.async_copy(edges.at[0, pl.ds(e0 + j * BLK, BLK)],
                             sib[k], isem[k])
            pltpu.async_copy(edges.at[1, pl.ds(e0 + j * BLK, BLK)],
                             dib[k], isem[k])

        def iwait(j, k):
            pltpu.make_async_copy(edges.at[0, pl.ds(e0 + j * BLK, BLK)],
                                  sib[k], isem[k]).wait()
            pltpu.make_async_copy(edges.at[1, pl.ds(e0 + j * BLK, BLK)],
                                  dib[k], isem[k]).wait()

        def gather(k, buf, sem):
            pltpu.async_copy(table.at[sib[k]], buf, sem)

        def gwait(k, buf, sem):
            pltpu.make_async_copy(table.at[sib[k]], buf, sem).wait()

        def scat(k, buf):
            pltpu.sync_copy(buf, acc_sh.at[dib[k]], add=True)
            if with_deg:
                pltpu.sync_copy(ones_v, deg_sh.at[dib[k]], add=True)

        pltpu.sync_copy(edges.at[0, pl.ds(e0, BLK)], sib[0])
        pltpu.sync_copy(edges.at[1, pl.ds(e0, BLK)], dib[0])
        gather(0, rows0, gs0)
        icopy(1, 1)
        icopy(2, 2)
        icopy(3, 3)
        plsc.subcore_barrier()

        def quad(jj, carry):
            b0 = 4 * jj
            iwait(b0 + 1, 1)
            gather(1, rows1, gs1)
            gwait(0, rows0, gs0)
            scat(0, rows0)
            icopy(b0 + 4, 0)

            iwait(b0 + 2, 2)
            gather(2, rows0, gs0)
            gwait(1, rows1, gs1)
            scat(1, rows1)

            @pl.when(b0 + 5 < NBLK)
            def _():
                icopy(b0 + 5, 1)

            iwait(b0 + 3, 3)
            gather(3, rows1, gs1)
            gwait(2, rows0, gs0)
            scat(2, rows0)

            @pl.when(b0 + 6 < NBLK)
            def _():
                icopy(b0 + 6, 2)

            iwait(b0 + 4, 0)
            gather(0, rows0, gs0)
            gwait(3, rows1, gs1)
            scat(3, rows1)

            @pl.when(b0 + 7 < NBLK)
            def _():
                icopy(b0 + 7, 3)

            return carry

        lax.fori_loop(0, NBLK // 4, quad, 0)
        gwait(0, rows0, gs0)
        scat(0, rows0)
        plsc.subcore_barrier()
        pltpu.sync_copy(acc_sh.at[pl.ds(r0, RPT)], out_p.at[c, pl.ds(r0, RPT)])
        if with_deg:
            pltpu.sync_copy(deg_sh.at[pl.ds(r0, RPT)],
                            out_d.at[c, pl.ds(r0, RPT)])

    return pl.kernel(body, out_type=out_type, mesh=mesh,
                     scratch_types=scratch,
                     compiler_params=pltpu.CompilerParams(
                         use_tc_tiling_on_sc=False))


_agg_deg = _make_agg(True)
_agg = _make_agg(False)


def _k1_body(x_ref, wn_ref, ws_ref, b_ref, a_ref, z_ref):
    xb = x_ref[...]
    a_ref[...] = jnp.dot(xb, wn_ref[...], preferred_element_type=jnp.float32)
    z_ref[...] = (jnp.dot(xb, ws_ref[...], preferred_element_type=jnp.float32)
                  + b_ref[...])


_k1 = pl.pallas_call(
    _k1_body,
    grid=(N // BR,),
    in_specs=[
        pl.BlockSpec((BR, D), lambda i: (i, 0)),
        pl.BlockSpec((D, D), lambda i: (0, 0)),
        pl.BlockSpec((D, D), lambda i: (0, 0)),
        pl.BlockSpec((1, D), lambda i: (0, 0)),
    ],
    out_specs=[
        pl.BlockSpec((BR, D), lambda i: (i, 0)),
        pl.BlockSpec((BR, D), lambda i: (i, 0)),
    ],
    out_shape=[
        jax.ShapeDtypeStruct((N, D), jnp.float32),
        jax.ShapeDtypeStruct((N, D), jnp.float32),
    ],
)


def _k2_body(p_ref, dg_ref, z1_ref, wn_ref, ws_ref, b_ref, a2_ref, z2_ref):
    s1 = p_ref[0] + p_ref[1]
    deg = jnp.maximum(dg_ref[0, :, :1] + dg_ref[1, :, :1], 1.0)
    h = jnp.maximum(z1_ref[...] + s1 / deg, 0.0)
    a2_ref[...] = jnp.dot(h, wn_ref[...], preferred_element_type=jnp.float32)
    z2_ref[...] = (jnp.dot(h, ws_ref[...], preferred_element_type=jnp.float32)
                   + b_ref[...])


_k2 = pl.pallas_call(
    _k2_body,
    grid=(N // BR,),
    in_specs=[
        pl.BlockSpec((NC, BR, D), lambda i: (0, i, 0)),
        pl.BlockSpec((NC, BR, DEGW), lambda i: (0, i, 0)),
        pl.BlockSpec((BR, D), lambda i: (i, 0)),
        pl.BlockSpec((D, D), lambda i: (0, 0)),
        pl.BlockSpec((D, D), lambda i: (0, 0)),
        pl.BlockSpec((1, D), lambda i: (0, 0)),
    ],
    out_specs=[
        pl.BlockSpec((BR, D), lambda i: (i, 0)),
        pl.BlockSpec((BR, D), lambda i: (i, 0)),
    ],
    out_shape=[
        jax.ShapeDtypeStruct((N, D), jnp.float32),
        jax.ShapeDtypeStruct((N, D), jnp.float32),
    ],
)


def _k3_body(p2_ref, dg_ref, z2_ref, out_ref):
    s2 = p2_ref[0] + p2_ref[1]
    deg = jnp.maximum(dg_ref[0, :, :1] + dg_ref[1, :, :1], 1.0)
    out_ref[...] = z2_ref[...] + s2 / deg


_k3 = pl.pallas_call(
    _k3_body,
    grid=(N // BR,),
    in_specs=[
        pl.BlockSpec((NC, BR, D), lambda i: (0, i, 0)),
        pl.BlockSpec((NC, BR, DEGW), lambda i: (0, i, 0)),
        pl.BlockSpec((BR, D), lambda i: (i, 0)),
    ],
    out_specs=pl.BlockSpec((BR, D), lambda i: (i, 0)),
    out_shape=jax.ShapeDtypeStruct((N, D), jnp.float32),
)


def kernel(x, edge_index, W_self1, W_neigh1, b1, W_self2, W_neigh2, b2):
    ei = edge_index if edge_index.dtype == jnp.int32 else (
        edge_index.astype(jnp.int32))
    b1r = b1.reshape(1, D)
    b2r = b2.reshape(1, D)

    a1, z1 = _k1(x, W_neigh1, W_self1, b1r)
    p1, dg = _agg_deg(a1, ei)
    a2, z2 = _k2(p1, dg, z1, W_neigh2, W_self2, b2r)
    (p2,) = _agg(a2, ei)
    return _k3(p2, dg, z2)


# speedup vs baseline: 4.5172x; 1.0041x over previous
import jax
import jax.numpy as jnp
from jax import lax
from jax.experimental import pallas as pl
from jax.experimental.pallas import tpu as pltpu
from jax.experimental.pallas import tpu_sc as plsc

N = 10000
E = 320000
D = 128
NC = 2
NS = 16
NW = NC * NS
BLK = 80
NBLK = 125
EPT = NBLK * BLK
NP = N
RPT = NP // NS
DEGW = 16
BR = 2000


def _make_agg(with_deg):
    mesh = plsc.VectorSubcoreMesh(core_axis_name="c", subcore_axis_name="s")
    out_type = [jax.ShapeDtypeStruct((NC, NP, D), jnp.float32)]
    scratch = (
        [pltpu.VMEM((BLK,), jnp.int32) for _ in range(4)]
        + [pltpu.VMEM((BLK,), jnp.int32) for _ in range(4)]
        + [
            pltpu.VMEM((BLK, D), jnp.float32),
            pltpu.VMEM((BLK, D), jnp.float32),
            pltpu.VMEM_SHARED((NP, D), jnp.float32),
        ]
        + [pltpu.SemaphoreType.DMA for _ in range(4)]
        + [pltpu.SemaphoreType.DMA for _ in range(2)]
    )
    if with_deg:
        out_type.append(jax.ShapeDtypeStruct((NC, NP, DEGW), jnp.float32))
        scratch += [
            pltpu.VMEM((BLK, DEGW), jnp.float32),
            pltpu.VMEM((BLK, DEGW), jnp.float32),
            pltpu.VMEM_SHARED((NP, DEGW), jnp.float32),
        ]

    assert NBLK % 4 == 1 and NBLK >= 5
    NFULL, NREM = RPT // BLK, RPT % BLK

    def body(*refs):
        if with_deg:
            (table, edges, out_p, out_d, *rest) = refs
            (s0, s1_, s2_, s3_, d0, d1_, d2_, d3_, rows0, rows1, acc_sh,
             is0, is1, is2, is3, gs0, gs1, ones_v, zd_v, deg_sh) = rest
        else:
            (table, edges, out_p, *rest) = refs
            (s0, s1_, s2_, s3_, d0, d1_, d2_, d3_, rows0, rows1, acc_sh,
             is0, is1, is2, is3, gs0, gs1) = rest
        sib = (s0, s1_, s2_, s3_)
        dib = (d0, d1_, d2_, d3_)
        isem = (is0, is1, is2, is3)
        c = lax.axis_index("c")
        s = lax.axis_index("s")
        wid = s * NC + c
        e0 = wid * EPT
        r0 = s * RPT

        zval = jnp.zeros((16,), jnp.float32)

        def zrow(i, carry):
            for kk in range(D // 16):
                rows1[i, pl.ds(16 * kk, 16)] = zval
            if with_deg:
                ones_v[i, :] = jnp.ones((16,), jnp.float32)
                zd_v[i, :] = zval
            return carry

        lax.fori_loop(0, BLK, zrow, 0)

        def icopy(j, k):
            pltpu.async_copy(edges.at[0, pl.ds(e0 + j * BLK, BLK)],
                             sib[k], isem[k])
            pltpu.async_copy(edges.at[1, pl.ds(e0 + j * BLK, BLK)],
                             dib[k], isem[k])

        def iwait(j, k):
            pltpu.make_async_copy(edges.at[0, pl.ds(e0 + j * BLK, BLK)],
                                  sib[k], isem[k]).wait()
            pltpu.make_async_copy(edges.at[1, pl.ds(e0 + j * BLK, BLK)],
                                  dib[k], isem[k]).wait()

        def gather(k, buf, sem):
            pltpu.async_copy(table.at[sib[k]], buf, sem)

        def gwait(k, buf, sem):
            pltpu.make_async_copy(table.at[sib[k]], buf, sem).wait()

        def scat(k, buf):
            pltpu.sync_copy(buf, acc_sh.at[dib[k]], add=True)
            if with_deg:
                pltpu.sync_copy(ones_v, deg_sh.at[dib[k]], add=True)

        pltpu.sync_copy(edges.at[0, pl.ds(e0, BLK)], sib[0])
        pltpu.sync_copy(edges.at[1, pl.ds(e0, BLK)], dib[0])
        gather(0, rows0, gs0)
        icopy(1, 1)
        icopy(2, 2)
        icopy(3, 3)
        for t in range(NFULL):
            pltpu.sync_copy(rows1, acc_sh.at[pl.ds(r0 + t * BLK, BLK)])
            if with_deg:
                pltpu.sync_copy(zd_v, deg_sh.at[pl.ds(r0 + t * BLK, BLK)])
        if NREM:
            pltpu.sync_copy(rows1.at[pl.ds(0, NREM)],
                            acc_sh.at[pl.ds(r0 + NFULL * BLK, NREM)])
            if with_deg:
                pltpu.sync_copy(zd_v.at[pl.ds(0, NREM)],
                                deg_sh.at[pl.ds(r0 + NFULL * BLK, NREM)])
        plsc.subcore_barrier()

        def quad(jj, carry):
            b0 = 4 * jj
            iwait(b0 + 1, 1)
            gather(1, rows1, gs1)
            gwait(0, rows0, gs0)
            scat(0, rows0)
            icopy(b0 + 4, 0)

            iwait(b0 + 2, 2)
            gather(2, rows0, gs0)
            gwait(1, rows1, gs1)
            scat(1, rows1)

            @pl.when(b0 + 5 < NBLK)
            def _():
                icopy(b0 + 5, 1)

            iwait(b0 + 3, 3)
            gather(3, rows1, gs1)
            gwait(2, rows0, gs0)
            scat(2, rows0)

            @pl.when(b0 + 6 < NBLK)
            def _():
                icopy(b0 + 6, 2)

            iwait(b0 + 4, 0)
            gather(0, rows0, gs0)
            gwait(3, rows1, gs1)
            scat(3, rows1)

            @pl.when(b0 + 7 < NBLK)
            def _():
                icopy(b0 + 7, 3)

            return carry

        lax.fori_loop(0, NBLK // 4, quad, 0)
        gwait(0, rows0, gs0)
        scat(0, rows0)
        plsc.subcore_barrier()
        pltpu.sync_copy(acc_sh.at[pl.ds(r0, RPT)], out_p.at[c, pl.ds(r0, RPT)])
        if with_deg:
            pltpu.sync_copy(deg_sh.at[pl.ds(r0, RPT)],
                            out_d.at[c, pl.ds(r0, RPT)])

    return pl.kernel(body, out_type=out_type, mesh=mesh,
                     scratch_types=scratch,
                     compiler_params=pltpu.CompilerParams(
                         use_tc_tiling_on_sc=False))


_agg_deg = _make_agg(True)
_agg = _make_agg(False)


def _k1_body(x_ref, wn_ref, ws_ref, b_ref, a_ref, z_ref):
    xb = x_ref[...]
    a_ref[...] = jnp.dot(xb, wn_ref[...], preferred_element_type=jnp.float32)
    z_ref[...] = (jnp.dot(xb, ws_ref[...], preferred_element_type=jnp.float32)
                  + b_ref[...])


_k1 = pl.pallas_call(
    _k1_body,
    grid=(N // BR,),
    in_specs=[
        pl.BlockSpec((BR, D), lambda i: (i, 0)),
        pl.BlockSpec((D, D), lambda i: (0, 0)),
        pl.BlockSpec((D, D), lambda i: (0, 0)),
        pl.BlockSpec((1, D), lambda i: (0, 0)),
    ],
    out_specs=[
        pl.BlockSpec((BR, D), lambda i: (i, 0)),
        pl.BlockSpec((BR, D), lambda i: (i, 0)),
    ],
    out_shape=[
        jax.ShapeDtypeStruct((N, D), jnp.float32),
        jax.ShapeDtypeStruct((N, D), jnp.float32),
    ],
)


def _k2_body(p_ref, dg_ref, z1_ref, wn_ref, ws_ref, b_ref, a2_ref, z2_ref):
    s1 = p_ref[0] + p_ref[1]
    deg = jnp.maximum(dg_ref[0, :, :1] + dg_ref[1, :, :1], 1.0)
    h = jnp.maximum(z1_ref[...] + s1 / deg, 0.0)
    a2_ref[...] = jnp.dot(h, wn_ref[...], preferred_element_type=jnp.float32)
    z2_ref[...] = (jnp.dot(h, ws_ref[...], preferred_element_type=jnp.float32)
                   + b_ref[...])


_k2 = pl.pallas_call(
    _k2_body,
    grid=(N // BR,),
    in_specs=[
        pl.BlockSpec((NC, BR, D), lambda i: (0, i, 0)),
        pl.BlockSpec((NC, BR, DEGW), lambda i: (0, i, 0)),
        pl.BlockSpec((BR, D), lambda i: (i, 0)),
        pl.BlockSpec((D, D), lambda i: (0, 0)),
        pl.BlockSpec((D, D), lambda i: (0, 0)),
        pl.BlockSpec((1, D), lambda i: (0, 0)),
    ],
    out_specs=[
        pl.BlockSpec((BR, D), lambda i: (i, 0)),
        pl.BlockSpec((BR, D), lambda i: (i, 0)),
    ],
    out_shape=[
        jax.ShapeDtypeStruct((N, D), jnp.float32),
        jax.ShapeDtypeStruct((N, D), jnp.float32),
    ],
)


def _k3_body(p2_ref, dg_ref, z2_ref, out_ref):
    s2 = p2_ref[0] + p2_ref[1]
    deg = jnp.maximum(dg_ref[0, :, :1] + dg_ref[1, :, :1], 1.0)
    out_ref[...] = z2_ref[...] + s2 / deg


_k3 = pl.pallas_call(
    _k3_body,
    grid=(N // BR,),
    in_specs=[
        pl.BlockSpec((NC, BR, D), lambda i: (0, i, 0)),
        pl.BlockSpec((NC, BR, DEGW), lambda i: (0, i, 0)),
        pl.BlockSpec((BR, D), lambda i: (i, 0)),
    ],
    out_specs=pl.BlockSpec((BR, D), lambda i: (i, 0)),
    out_shape=jax.ShapeDtypeStruct((N, D), jnp.float32),
)


def kernel(x, edge_index, W_self1, W_neigh1, b1, W_self2, W_neigh2, b2):
    ei = edge_index if edge_index.dtype == jnp.int32 else (
        edge_index.astype(jnp.int32))
    b1r = b1.reshape(1, D)
    b2r = b2.reshape(1, D)

    a1, z1 = _k1(x, W_neigh1, W_self1, b1r)
    p1, dg = _agg_deg(a1, ei)
    a2, z2 = _k2(p1, dg, z1, W_neigh2, W_self2, b2r)
    (p2,) = _agg(a2, ei)
    return _k3(p2, dg, z2)
